# baseline (speedup 1.0000x reference)
import jax
import jax.numpy as jnp
from jax import lax
from jax.experimental import pallas as pl
from jax.experimental.pallas import tpu as pltpu

N = 32768
S1 = 512
S2 = 32
PAD = 64
BLK = 2048
NBLK = N // BLK
PBLK = 256
H2 = 256
H3 = 512
D = 384
PH = 128

_NEG = float("-inf")
_BF = jnp.bfloat16


def _seg_max_scan(v, ids):
    row = lax.broadcasted_iota(jnp.int32, (BLK, 1), 0)
    k = 1
    while k < BLK:
        v_sh = pltpu.roll(v, k, axis=0)
        id_sh = pltpu.roll(ids, k, axis=0)
        ok = (row >= k) & (ids == id_sh)
        pen = jnp.where(ok, 0.0, -1e30).astype(_BF)
        v = jnp.maximum(v, v_sh + pen)
        k *= 2
    return v


def _pick_last_row(v, fill):
    rid = lax.broadcasted_iota(jnp.int32, v.shape, 0)
    return jnp.max(jnp.where(rid == v.shape[0] - 1, v, fill), axis=0, keepdims=True)


def _carry_fix(vs, ids, cval_ref, cid_ref, c):
    cid = cid_ref[0:1, 0:1]
    cval = cval_ref[0:1, 0:c]
    vs = jnp.where(ids == cid, jnp.maximum(vs, cval), vs)
    cval_ref[0:1, 0:c] = _pick_last_row(vs, _NEG)
    cid_ref[0:1, 0:1] = _pick_last_row(ids, -1)[:, 0:1]
    return vs


def _body(x_ref, ids_ref, idn_ref, wct_ref, bc_ref, wf_ref,
          at_ref, b2af_ref, w2bt_ref, b2b_ref, coords_ref, wp1t_ref, bp1_ref,
          wp2t_ref, bp2_ref, s21_ref, tokp_ref, posp_ref,
          g_ref, gseg_ref, tacc_ref, cval_ref, cid_ref):
    b = pl.program_id(0)

    @pl.when(b == 0)
    def _():
        g_ref[...] = jnp.zeros_like(g_ref)
        cid_ref[...] = jnp.full_like(cid_ref, -1)

    @pl.when(b < NBLK)
    def _phase1():
        ids = ids_ref[...]
        idn = idn_ref[...]
        ohfin = ((ids == lax.broadcasted_iota(jnp.int32, (BLK, S1), 1))
                 & (ids != idn)).astype(_BF)
        h = jnp.dot(x_ref[...], wct_ref[...],
                    preferred_element_type=jnp.float32) + bc_ref[...]
        hs = _seg_max_scan(h.astype(_BF), ids)
        hs = _carry_fix(hs, ids, cval_ref, cid_ref, H2)
        contrib = lax.dot_general(ohfin, hs, (((0,), (0,)), ((), ())),
                                  preferred_element_type=jnp.float32)
        g_ref[...] += contrib.astype(_BF)

    @pl.when(b == NBLK)
    def _():
        gseg_ref[...] = (jnp.dot(g_ref[...], at_ref[...],
                                 preferred_element_type=jnp.float32)
                         + b2af_ref[...]).astype(_BF)
        tacc_ref[...] = jnp.zeros_like(tacc_ref)
        cid_ref[...] = jnp.full_like(cid_ref, -1)

    @pl.when(b >= NBLK)
    def _phase2():
        ids = ids_ref[...]
        idn = idn_ref[...]
        ohb = ids == lax.broadcasted_iota(jnp.int32, (BLK, S1), 1)
        ohfin = (ohb & (ids != idn)).astype(_BF)
        onehot = ohb.astype(_BF)
        expand = jnp.dot(onehot, gseg_ref[...], preferred_element_type=jnp.float32)
        t = jnp.maximum(jnp.dot(x_ref[...], wf_ref[...],
                                preferred_element_type=jnp.float32) + expand, 0.0)
        u = jnp.dot(t.astype(_BF), w2bt_ref[...],
                    preferred_element_type=jnp.float32) + b2b_ref[...]
        us = _seg_max_scan(u.astype(_BF), ids)
        us = _carry_fix(us, ids, cval_ref, cid_ref, D)
        tacc_ref[...] += lax.dot_general(
            ohfin, us, (((0,), (0,)), ((), ())),
            preferred_element_type=jnp.float32).astype(_BF)

    @pl.when(b == 2 * NBLK - 1)
    def _tail():
        tokens = tacc_ref[...]
        z = jnp.dot(coords_ref[...], wp1t_ref[...],
                    preferred_element_type=jnp.float32) + bp1_ref[...]
        z = 0.5 * z * (1.0 + lax.erf(z * 0.7071067811865476))
        pos = jnp.dot(z, wp2t_ref[...], preferred_element_type=jnp.float32) + bp2_ref[...]
        s21c = s21_ref[...]
        j32 = lax.broadcasted_iota(jnp.int32, (S1, S2), 1)
        start2 = jnp.sum((s21c < j32).astype(jnp.int32), axis=0, keepdims=True)
        end2 = jnp.sum((s21c <= j32).astype(jnp.int32), axis=0, keepdims=True)
        for c in range(8):
            drow = lax.broadcasted_iota(jnp.int32, (PBLK, 1), 0) + c * PBLK
            s2 = drow // PAD
            r = drow % PAD
            oh2 = s2 == lax.broadcasted_iota(jnp.int32, (PBLK, S2), 1)
            start_d = jnp.sum(jnp.where(oh2, start2, 0), axis=1, keepdims=True)
            end_d = jnp.sum(jnp.where(oh2, end2, 0), axis=1, keepdims=True)
            src = start_d + r
            valid = src < end_d
            ohp = ((src == lax.broadcasted_iota(jnp.int32, (PBLK, S1), 1)) & valid
                   ).astype(_BF)
            sl = pl.ds(c * PBLK, PBLK)
            tokp_ref[sl, :] = jnp.dot(ohp, tokens, preferred_element_type=jnp.float32)
            posp_ref[sl, :] = jnp.dot(ohp, pos.astype(_BF),
                                      preferred_element_type=jnp.float32)


def kernel(full_features, sp_coords, full_super_indices_10, full_super_indices_21,
           W1a, b1a, W1b, b1b, W2a, b2a, W2b, b2b, Wp1, bp1, Wp2, bp2):
    x = full_features[0]
    coords = sp_coords[0]
    s10 = full_super_indices_10[0].astype(jnp.int32)
    s21 = full_super_indices_21[0].astype(jnp.int32)

    wct = (W1b @ W1a).T
    bc = (W1b @ b1a + b1b).reshape(1, H2)
    at = W2a[:, :H2].T.astype(_BF)
    bt = W2a[:, H2:].T
    wf = wct @ bt
    b2af = (bc @ bt + b2a.reshape(1, H3))
    w2bt = W2b.T.astype(_BF)
    b2b_r = b2b.reshape(1, D)
    wp1t = Wp1.T
    bp1_r = bp1.reshape(1, PH)
    wp2t = Wp2.T
    bp2_r = bp2.reshape(1, D)

    ids = s10.reshape(N, 1)
    idn = jnp.concatenate([s10[1:], jnp.full((1,), -1, jnp.int32)]).reshape(N, 1)
    s21c = s21.reshape(S1, 1)

    full = lambda shape: pl.BlockSpec(shape, lambda b: (0, 0))
    blk = lambda shape: pl.BlockSpec(shape, lambda b: (b % NBLK, 0))

    tokp, posp = pl.pallas_call(
        _body,
        grid=(2 * NBLK,),
        in_specs=[blk((BLK, 11)), blk((BLK, 1)), blk((BLK, 1)),
                  full((11, H2)), full((1, H2)), full((11, H3)),
                  full((H2, H3)), full((1, H3)),
                  full((H3, D)), full((1, D)),
                  full((S1, 3)), full((3, PH)), full((1, PH)),
                  full((PH, D)), full((1, D)), full((S1, 1))],
        out_specs=[full((S2 * PAD, D)), full((S2 * PAD, D))],
        out_shape=[jax.ShapeDtypeStruct((S2 * PAD, D), jnp.float32),
                   jax.ShapeDtypeStruct((S2 * PAD, D), jnp.float32)],
        scratch_shapes=[pltpu.VMEM((S1, H2), _BF),
                        pltpu.VMEM((S1, H3), _BF),
                        pltpu.VMEM((S1, D), _BF),
                        pltpu.VMEM((8, H3), _BF),
                        pltpu.VMEM((8, 128), jnp.int32)],
    )(x, ids, idn, wct, bc, wf, at, b2af, w2bt, b2b_r,
      coords, wp1t, bp1_r, wp2t, bp2_r, s21c)

    return (tokp.reshape(1, S2, PAD, D), posp.reshape(1, S2, PAD, D))


# speedup vs baseline: 1.0366x; 1.0366x over previous
import jax
import jax.numpy as jnp
from jax import lax
from jax.experimental import pallas as pl
from jax.experimental.pallas import tpu as pltpu

N = 32768
S1 = 512
S2 = 32
PAD = 64
BLK = 1024
NBLK = N // BLK
PBLK = 256
H2 = 256
H3 = 512
D = 384
PH = 128

_NEG = float("-inf")
_BF = jnp.bfloat16


def _seg_max_scan(v, ids):
    row = lax.broadcasted_iota(jnp.int32, (BLK, 1), 0)
    k = 1
    while k < BLK:
        v_sh = pltpu.roll(v, k, axis=0)
        id_sh = pltpu.roll(ids, k, axis=0)
        ok = (row >= k) & (ids == id_sh)
        pen = jnp.where(ok, 0.0, -1e30).astype(_BF)
        v = jnp.maximum(v, v_sh + pen)
        k *= 2
    return v


def _pick_last_row(v, fill):
    rid = lax.broadcasted_iota(jnp.int32, v.shape, 0)
    return jnp.max(jnp.where(rid == v.shape[0] - 1, v, fill), axis=0, keepdims=True)


def _carry_fix(vs, ids, cval_ref, cid_ref, c):
    cid = cid_ref[0:1, 0:1]
    cval = cval_ref[0:1, 0:c]
    vs = jnp.where(ids == cid, jnp.maximum(vs, cval), vs)
    cval_ref[0:1, 0:c] = _pick_last_row(vs, _NEG)
    cid_ref[0:1, 0:1] = _pick_last_row(ids, -1)[:, 0:1]
    return vs


def _body(x_ref, ids_ref, idn_ref, wct_ref, bc_ref, wf_ref,
          at_ref, b2af_ref, w2bt_ref, b2b_ref, coords_ref, wp1t_ref, bp1_ref,
          wp2t_ref, bp2_ref, s21_ref, tokp_ref, posp_ref,
          g_ref, gseg_ref, tacc_ref, cval_ref, cid_ref):
    b = pl.program_id(0)

    @pl.when(b == 0)
    def _():
        g_ref[...] = jnp.zeros_like(g_ref)
        cid_ref[...] = jnp.full_like(cid_ref, -1)

    @pl.when(b < NBLK)
    def _phase1():
        ids = ids_ref[...]
        idn = idn_ref[...]
        ohfin = ((ids == lax.broadcasted_iota(jnp.int32, (BLK, S1), 1))
                 & (ids != idn)).astype(_BF)
        h = jnp.dot(x_ref[...], wct_ref[...],
                    preferred_element_type=jnp.float32) + bc_ref[...]
        hs = _seg_max_scan(h.astype(_BF), ids)
        hs = _carry_fix(hs, ids, cval_ref, cid_ref, H2)
        contrib = lax.dot_general(ohfin, hs, (((0,), (0,)), ((), ())),
                                  preferred_element_type=jnp.float32)
        g_ref[...] += contrib.astype(_BF)

    @pl.when(b == NBLK)
    def _():
        gseg_ref[...] = (jnp.dot(g_ref[...], at_ref[...],
                                 preferred_element_type=jnp.float32)
                         + b2af_ref[...]).astype(_BF)
        tacc_ref[...] = jnp.zeros_like(tacc_ref)
        cid_ref[...] = jnp.full_like(cid_ref, -1)

    @pl.when(b >= NBLK)
    def _phase2():
        ids = ids_ref[...]
        idn = idn_ref[...]
        onehot = (ids == lax.broadcasted_iota(jnp.int32, (BLK, S1), 1)).astype(_BF)
        ohfin = onehot * (ids != idn).astype(_BF)
        expand = jnp.dot(onehot, gseg_ref[...], preferred_element_type=jnp.float32)
        t = jnp.maximum(jnp.dot(x_ref[...], wf_ref[...],
                                preferred_element_type=jnp.float32) + expand, 0.0)
        u = jnp.dot(t.astype(_BF), w2bt_ref[...],
                    preferred_element_type=jnp.float32) + b2b_ref[...]
        us = _seg_max_scan(u.astype(_BF), ids)
        us = _carry_fix(us, ids, cval_ref, cid_ref, D)
        tacc_ref[...] += lax.dot_general(
            ohfin, us, (((0,), (0,)), ((), ())),
            preferred_element_type=jnp.float32).astype(_BF)

    @pl.when(b == 2 * NBLK - 1)
    def _tail():
        tokens = tacc_ref[...]
        z = jnp.dot(coords_ref[...], wp1t_ref[...],
                    preferred_element_type=jnp.float32) + bp1_ref[...]
        z = 0.5 * z * (1.0 + lax.erf(z * 0.7071067811865476))
        pos = jnp.dot(z, wp2t_ref[...], preferred_element_type=jnp.float32) + bp2_ref[...]
        s21c = s21_ref[...]
        j32 = lax.broadcasted_iota(jnp.int32, (S1, S2), 1)
        start2 = jnp.sum((s21c < j32).astype(jnp.int32), axis=0, keepdims=True)
        end2 = jnp.sum((s21c <= j32).astype(jnp.int32), axis=0, keepdims=True)
        for c in range(8):
            drow = lax.broadcasted_iota(jnp.int32, (PBLK, 1), 0) + c * PBLK
            s2 = drow // PAD
            r = drow % PAD
            oh2 = s2 == lax.broadcasted_iota(jnp.int32, (PBLK, S2), 1)
            start_d = jnp.sum(jnp.where(oh2, start2, 0), axis=1, keepdims=True)
            end_d = jnp.sum(jnp.where(oh2, end2, 0), axis=1, keepdims=True)
            src = start_d + r
            valid = src < end_d
            ohp = ((src == lax.broadcasted_iota(jnp.int32, (PBLK, S1), 1)) & valid
                   ).astype(_BF)
            sl = pl.ds(c * PBLK, PBLK)
            tokp_ref[sl, :] = jnp.dot(ohp, tokens, preferred_element_type=jnp.float32)
            posp_ref[sl, :] = jnp.dot(ohp, pos.astype(_BF),
                                      preferred_element_type=jnp.float32)


def kernel(full_features, sp_coords, full_super_indices_10, full_super_indices_21,
           W1a, b1a, W1b, b1b, W2a, b2a, W2b, b2b, Wp1, bp1, Wp2, bp2):
    x = full_features[0]
    coords = sp_coords[0]
    s10 = full_super_indices_10[0].astype(jnp.int32)
    s21 = full_super_indices_21[0].astype(jnp.int32)

    wct = (W1b @ W1a).T
    bc = (W1b @ b1a + b1b).reshape(1, H2)
    at = W2a[:, :H2].T.astype(_BF)
    bt = W2a[:, H2:].T
    wf = wct @ bt
    b2af = (bc @ bt + b2a.reshape(1, H3))
    w2bt = W2b.T.astype(_BF)
    b2b_r = b2b.reshape(1, D)
    wp1t = Wp1.T
    bp1_r = bp1.reshape(1, PH)
    wp2t = Wp2.T
    bp2_r = bp2.reshape(1, D)

    ids = s10.reshape(N, 1)
    idn = jnp.concatenate([s10[1:], jnp.full((1,), -1, jnp.int32)]).reshape(N, 1)
    s21c = s21.reshape(S1, 1)

    full = lambda shape: pl.BlockSpec(shape, lambda b: (0, 0))
    blk = lambda shape: pl.BlockSpec(shape, lambda b: (b % NBLK, 0))

    tokp, posp = pl.pallas_call(
        _body,
        grid=(2 * NBLK,),
        in_specs=[blk((BLK, 11)), blk((BLK, 1)), blk((BLK, 1)),
                  full((11, H2)), full((1, H2)), full((11, H3)),
                  full((H2, H3)), full((1, H3)),
                  full((H3, D)), full((1, D)),
                  full((S1, 3)), full((3, PH)), full((1, PH)),
                  full((PH, D)), full((1, D)), full((S1, 1))],
        out_specs=[full((S2 * PAD, D)), full((S2 * PAD, D))],
        out_shape=[jax.ShapeDtypeStruct((S2 * PAD, D), jnp.float32),
                   jax.ShapeDtypeStruct((S2 * PAD, D), jnp.float32)],
        scratch_shapes=[pltpu.VMEM((S1, H2), _BF),
                        pltpu.VMEM((S1, H3), _BF),
                        pltpu.VMEM((S1, D), _BF),
                        pltpu.VMEM((8, H3), _BF),
                        pltpu.VMEM((8, 128), jnp.int32)],
    )(x, ids, idn, wct, bc, wf, at, b2af, w2bt, b2b_r,
      coords, wp1t, bp1_r, wp2t, bp2_r, s21c)

    return (tokp.reshape(1, S2, PAD, D), posp.reshape(1, S2, PAD, D))
